# Initial kernel scaffold; baseline (speedup 1.0000x reference)
import functools

import jax
import jax.numpy as jnp
from jax import lax
from jax.experimental import pallas as pl
from jax.experimental.pallas import tpu as pltpu
from jax.experimental.pallas import tpu_sc as plsc

_NUM_CORES = 2
_NUM_SUBCORES = 16
_NW = _NUM_CORES * _NUM_SUBCORES
_D = 32
_CHUNK = 1024


@functools.cache
def _make_gather(B: int):
    assert B % (_NW * _CHUNK) == 0
    b_per_w = B // _NW
    n_chunks = b_per_w // _CHUNK
    mesh = plsc.VectorSubcoreMesh(core_axis_name="c", subcore_axis_name="s")

    @functools.partial(
        pl.kernel,
        out_type=jax.ShapeDtypeStruct((B, _D), jnp.float32),
        mesh=mesh,
        scratch_types=[
            pltpu.VMEM((_CHUNK,), jnp.int32),
            pltpu.VMEM((_CHUNK, _D), jnp.float32),
            pltpu.SemaphoreType.DMA,
        ],
    )
    def gather_kernel(table_hbm, idx_hbm, out_hbm, idx_v, rows_v, sem):
        wid = lax.axis_index("s") * _NUM_CORES + lax.axis_index("c")
        base = wid * b_per_w

        def body(i, carry):
            off = base + i * _CHUNK
            pltpu.sync_copy(idx_hbm.at[pl.ds(off, _CHUNK)], idx_v)
            pltpu.async_copy(table_hbm.at[idx_v], rows_v, sem).wait()
            pltpu.sync_copy(rows_v, out_hbm.at[pl.ds(off, _CHUNK)])
            return carry

        lax.fori_loop(0, n_chunks, body, 0)

    return gather_kernel


def kernel(x, table):
    batch, hist = x.shape
    B = batch * hist
    idx = x.reshape(B).astype(jnp.int32)
    out = _make_gather(B)(table, idx)
    return out.reshape(batch, hist, _D)


# speedup vs baseline: 1.0947x; 1.0947x over previous
import functools

import jax
import jax.numpy as jnp
from jax import lax
from jax.experimental import pallas as pl
from jax.experimental.pallas import tpu as pltpu
from jax.experimental.pallas import tpu_sc as plsc

_NUM_CORES = 2
_NUM_SUBCORES = 16
_NW = _NUM_CORES * _NUM_SUBCORES
_D = 32
_CHUNK = 1024


@functools.cache
def _make_gather(B: int):
    assert B % (_NW * _CHUNK) == 0
    b_per_w = B // _NW
    n_chunks = b_per_w // _CHUNK
    mesh = plsc.VectorSubcoreMesh(core_axis_name="c", subcore_axis_name="s")

    @functools.partial(
        pl.kernel,
        out_type=jax.ShapeDtypeStruct((B, _D), jnp.float32),
        mesh=mesh,
        scratch_types=[
            pltpu.VMEM((_CHUNK,), jnp.int32),
            pltpu.VMEM((_CHUNK, _D), jnp.float32),
            pltpu.SemaphoreType.DMA,
        ],
        compiler_params=pltpu.CompilerParams(use_tc_tiling_on_sc=False),
    )
    def gather_kernel(table_hbm, idx_hbm, out_hbm, idx_v, rows_v, sem):
        wid = lax.axis_index("s") * _NUM_CORES + lax.axis_index("c")
        base = wid * b_per_w

        def body(i, carry):
            off = base + i * _CHUNK
            pltpu.sync_copy(idx_hbm.at[pl.ds(off, _CHUNK)], idx_v)
            pltpu.async_copy(table_hbm.at[idx_v], rows_v, sem).wait()
            pltpu.sync_copy(rows_v, out_hbm.at[pl.ds(off, _CHUNK)])
            return carry

        lax.fori_loop(0, n_chunks, body, 0)

    return gather_kernel


def kernel(x, table):
    batch, hist = x.shape
    B = batch * hist
    idx = x.reshape(B).astype(jnp.int32)
    out = _make_gather(B)(table, idx)
    return out.reshape(batch, hist, _D)


# speedup vs baseline: 1.1139x; 1.0175x over previous
import functools

import jax
import jax.numpy as jnp
from jax import lax
from jax.experimental import pallas as pl
from jax.experimental.pallas import tpu as pltpu
from jax.experimental.pallas import tpu_sc as plsc

_NUM_CORES = 2
_NUM_SUBCORES = 16
_NW = _NUM_CORES * _NUM_SUBCORES
_D = 32
_CHUNK = 1280


@functools.cache
def _make_gather(B: int):
    assert B % (_NW * _CHUNK) == 0
    b_per_w = B // _NW
    n_chunks = b_per_w // _CHUNK
    assert n_chunks % 2 == 0 and n_chunks >= 4
    mesh = plsc.VectorSubcoreMesh(core_axis_name="c", subcore_axis_name="s")

    @functools.partial(
        pl.kernel,
        out_type=jax.ShapeDtypeStruct((B, _D), jnp.float32),
        mesh=mesh,
        scratch_types=[
            pltpu.VMEM((b_per_w,), jnp.int32),
            pltpu.VMEM((_CHUNK, _D), jnp.float32),
            pltpu.VMEM((_CHUNK, _D), jnp.float32),
            pltpu.SemaphoreType.DMA,
            pltpu.SemaphoreType.DMA,
        ],
        compiler_params=pltpu.CompilerParams(use_tc_tiling_on_sc=False),
    )
    def gather_kernel(table_hbm, idx_hbm, out_hbm, idx_v, rows0, rows1, sem0, sem1):
        wid = lax.axis_index("s") * _NUM_CORES + lax.axis_index("c")
        base = wid * b_per_w
        rows = (rows0, rows1)
        sems = (sem0, sem1)

        pltpu.sync_copy(idx_hbm.at[pl.ds(base, b_per_w)], idx_v)

        for b in range(2):
            pltpu.async_copy(
                table_hbm.at[idx_v.at[pl.ds(b * _CHUNK, _CHUNK)]], rows[b], sems[b]
            )

        def body(i2, carry):
            for b in range(2):
                i = 2 * i2 + b
                pltpu.make_async_copy(
                    table_hbm.at[idx_v.at[pl.ds(0, _CHUNK)]], rows[b], sems[b]
                ).wait()
                pltpu.sync_copy(rows[b], out_hbm.at[pl.ds(base + i * _CHUNK, _CHUNK)])
                pltpu.async_copy(
                    table_hbm.at[idx_v.at[pl.ds((i + 2) * _CHUNK, _CHUNK)]],
                    rows[b],
                    sems[b],
                )
            return carry

        lax.fori_loop(0, n_chunks // 2 - 1, body, 0)

        for b in range(2):
            i = n_chunks - 2 + b
            pltpu.make_async_copy(
                table_hbm.at[idx_v.at[pl.ds(0, _CHUNK)]], rows[b], sems[b]
            ).wait()
            pltpu.sync_copy(rows[b], out_hbm.at[pl.ds(base + i * _CHUNK, _CHUNK)])

    return gather_kernel


def kernel(x, table):
    batch, hist = x.shape
    B = batch * hist
    idx = x.reshape(B).astype(jnp.int32)
    out = _make_gather(B)(table, idx)
    return out.reshape(batch, hist, _D)


# speedup vs baseline: 1.8080x; 1.6231x over previous
import functools

import jax
import jax.numpy as jnp
from jax import lax
from jax.experimental import pallas as pl
from jax.experimental.pallas import tpu as pltpu
from jax.experimental.pallas import tpu_sc as plsc

_NUM_CORES = 2
_NUM_SUBCORES = 16
_NW = _NUM_CORES * _NUM_SUBCORES
_D = 32
_CHUNK = 1600


@functools.cache
def _make_gather(batch: int, hist: int):
    B = batch * hist
    assert B % (_NW * _CHUNK) == 0
    b_per_w = B // _NW
    n_chunks = b_per_w // _CHUNK
    assert n_chunks % 2 == 0 and n_chunks >= 4
    nb = _CHUNK // hist
    mesh = plsc.VectorSubcoreMesh(core_axis_name="c", subcore_axis_name="s")

    @functools.partial(
        pl.kernel,
        out_type=jax.ShapeDtypeStruct((batch, hist, _D), jnp.float32),
        mesh=mesh,
        scratch_types=[
            pltpu.VMEM((b_per_w,), jnp.int32),
            pltpu.VMEM((_CHUNK, _D), jnp.float32),
            pltpu.VMEM((_CHUNK, _D), jnp.float32),
            pltpu.SemaphoreType.DMA,
            pltpu.SemaphoreType.DMA,
            pltpu.SemaphoreType.DMA,
            pltpu.SemaphoreType.DMA,
        ],
        compiler_params=pltpu.CompilerParams(use_tc_tiling_on_sc=False),
    )
    def gather_kernel(
        table_hbm, idx_hbm, out_hbm, idx_v, rows0, rows1, g0, g1, s0, s1
    ):
        wid = lax.axis_index("s") * _NUM_CORES + lax.axis_index("c")
        base = wid * b_per_w
        row_base = wid * (b_per_w // hist)
        rows = (rows0, rows1)
        gsem = (g0, g1)
        ssem = (s0, s1)

        pltpu.sync_copy(idx_hbm.at[pl.ds(base, b_per_w)], idx_v)

        def fire_gather(i, b):
            pltpu.async_copy(
                table_hbm.at[idx_v.at[pl.ds(i * _CHUNK, _CHUNK)]], rows[b], gsem[b]
            )

        def wait_gather(b):
            pltpu.make_async_copy(
                table_hbm.at[idx_v.at[pl.ds(0, _CHUNK)]], rows[b], gsem[b]
            ).wait()

        def fire_stores(i, b):
            for k in range(nb):
                pltpu.async_copy(
                    rows[b].at[pl.ds(k * hist, hist)],
                    out_hbm.at[row_base + i * nb + k],
                    ssem[b],
                )

        def drain_stores(b):
            for _ in range(nb):
                pltpu.make_async_copy(
                    rows[b].at[pl.ds(0, hist)], out_hbm.at[row_base], ssem[b]
                ).wait()

        for b in range(2):
            fire_gather(b, b)

        def body(i2, carry):
            for b in range(2):
                i = 2 * i2 + b
                wait_gather(b)
                fire_stores(i, b)
                drain_stores(b)
                fire_gather(i + 2, b)
            return carry

        lax.fori_loop(0, n_chunks // 2 - 1, body, 0)

        for b in range(2):
            wait_gather(b)
            fire_stores(n_chunks - 2 + b, b)
            drain_stores(b)

    return gather_kernel


def kernel(x, table):
    batch, hist = x.shape
    idx = x.reshape(batch * hist).astype(jnp.int32)
    return _make_gather(batch, hist)(table, idx)


# speedup vs baseline: 1.8551x; 1.0260x over previous
import functools

import jax
import jax.numpy as jnp
from jax import lax
from jax.experimental import pallas as pl
from jax.experimental.pallas import tpu as pltpu
from jax.experimental.pallas import tpu_sc as plsc

_NUM_CORES = 2
_NUM_SUBCORES = 16
_NW = _NUM_CORES * _NUM_SUBCORES
_D = 32
_LANES = 16
_TILE_B = 128
_NBUF = 3


@functools.cache
def _make_gather(batch: int, hist: int):
    B = batch * hist
    bw = batch // _NW
    nbt = bw // _TILE_B
    ncg = _D // 8
    t1_len = bw * _D
    slab = _D * batch
    mesh = plsc.VectorSubcoreMesh(core_axis_name="c", subcore_axis_name="s")

    @functools.partial(
        pl.kernel,
        out_type=jax.ShapeDtypeStruct((B * _D,), jnp.float32),
        mesh=mesh,
        scratch_types=[
            pltpu.VMEM((hist, bw), jnp.int32),
            pltpu.VMEM((bw, _D), jnp.float32),
            pltpu.VMEM((bw, _D), jnp.float32),
            pltpu.VMEM((bw, _D), jnp.float32),
            pltpu.VMEM((t1_len,), jnp.float32),
            pltpu.VMEM((t1_len,), jnp.float32),
            pltpu.SemaphoreType.DMA,
            pltpu.SemaphoreType.DMA,
            pltpu.SemaphoreType.DMA,
            pltpu.SemaphoreType.DMA,
            pltpu.SemaphoreType.DMA,
            pltpu.SemaphoreType.DMA,
        ],
        compiler_params=pltpu.CompilerParams(
            use_tc_tiling_on_sc=False, needs_layout_passes=False
        ),
    )
    def gather_kernel(
        table_hbm, idx_hbm, out_hbm,
        idx_v, r0, r1, r2, t0, t1,
        g0, g1, g2, isem, s0, s1,
    ):
        wid = lax.axis_index("s") * _NUM_CORES + lax.axis_index("c")
        col0 = wid * bw
        rows = (r0, r1, r2)
        gsem = (g0, g1, g2)
        tiles = (t0, t1)
        ssem = (s0, s1)

        for h in range(hist):
            pltpu.async_copy(
                idx_hbm.at[pl.ds(h * batch + col0, bw)], idx_v.at[h], isem
            )
        for h in range(hist):
            pltpu.make_async_copy(
                idx_hbm.at[pl.ds(0, bw)], idx_v.at[0], isem
            ).wait()

        lane = lax.iota(jnp.int32, _LANES)
        cvec0 = lane * 128
        cvec1 = cvec0 + 2048

        def fire_gather(h, rb):
            pltpu.async_copy(
                table_hbm.at[idx_v.at[h]], rows[rb], gsem[rb]
            )

        def wait_gather(rb):
            pltpu.make_async_copy(
                table_hbm.at[idx_v.at[0]], rows[rb], gsem[rb]
            ).wait()

        def transpose(rb, tb):
            src = rows[rb]
            dst = tiles[tb]

            def tr_body(i, carry):
                for j in range(8):
                    b = i * 8 + j
                    boff = (b >> 7) * (ncg * 8 * 128) + (b & 127)
                    v0 = src[b, pl.ds(0, _LANES)]
                    v1 = src[b, pl.ds(_LANES, _LANES)]
                    plsc.store_scatter(dst, [cvec0 + boff], v0)
                    plsc.store_scatter(dst, [cvec1 + boff], v1)
                return carry

            lax.fori_loop(0, bw // 8, tr_body, 0)

        def fire_stores(h, tb):
            for bt in range(nbt):
                for cg in range(ncg):
                    pltpu.async_copy(
                        tiles[tb].at[pl.ds((bt * ncg + cg) * 1024, 1024)],
                        out_hbm.at[
                            pl.ds(
                                h * slab + cg * (batch * 8)
                                + (wid * nbt + bt) * 1024,
                                1024,
                            )
                        ],
                        ssem[tb],
                    )

        def drain_stores(tb):
            for _ in range(nbt * ncg):
                pltpu.make_async_copy(
                    tiles[tb].at[pl.ds(0, 1024)],
                    out_hbm.at[pl.ds(0, 1024)],
                    ssem[tb],
                ).wait()

        for p in range(_NBUF):
            fire_gather(p, p)

        def step(h, rb, tb, drain, fire):
            wait_gather(rb)
            if drain:
                drain_stores(tb)
            transpose(rb, tb)
            fire_stores(h, tb)
            if fire == "always":
                fire_gather(h + _NBUF, rb)
            elif fire == "cond":
                nxt = h + _NBUF

                @pl.when(nxt < hist)
                def _():
                    fire_gather(nxt, rb)

        for k in range(6):
            step(k, k % _NBUF, k % 2, drain=k >= 2, fire="always")

        def body(i2, carry):
            for k in range(6):
                h = 6 + i2 * 6 + k
                step(h, k % _NBUF, k % 2, drain=True, fire="cond")
            return carry

        n_main = (hist - 6) // 6
        lax.fori_loop(0, n_main, body, 0)

        for k in range(hist - 6 - n_main * 6):
            h = 6 + n_main * 6 + k
            step(h, h % _NBUF, h % 2, drain=True, fire="never")

        for tb in range(2):
            drain_stores(tb)

    return gather_kernel


def kernel(x, table):
    batch, hist = x.shape
    idx = x.T.reshape(batch * hist).astype(jnp.int32)
    flat = _make_gather(batch, hist)(table, idx)
    a = flat.reshape(hist, _D // 8, batch // _TILE_B, 8, _TILE_B)
    return a.transpose(2, 4, 0, 1, 3).reshape(batch, hist, _D)


# speedup vs baseline: 1.8758x; 1.0111x over previous
import functools

import jax
import jax.numpy as jnp
from jax import lax
from jax.experimental import pallas as pl
from jax.experimental.pallas import tpu as pltpu
from jax.experimental.pallas import tpu_sc as plsc

_NUM_CORES = 2
_NUM_SUBCORES = 16
_NW = _NUM_CORES * _NUM_SUBCORES
_D = 32
_LANES = 16
_TILE_B = 128
_NBUF = 2
_TW = 128


@functools.cache
def _make_gather(batch: int, hist: int):
    B = batch * hist
    bw = batch // _NW
    nbt = bw // _TILE_B
    ncg = _D // 8
    t1_len = bw * _D
    slab = _D * batch
    mesh = plsc.VectorSubcoreMesh(core_axis_name="c", subcore_axis_name="s")

    @functools.partial(
        pl.kernel,
        out_type=jax.ShapeDtypeStruct((B * _D,), jnp.float32),
        mesh=mesh,
        scratch_types=[
            pltpu.VMEM((hist, bw), jnp.int32),
            pltpu.VMEM((bw // 2, _TW), jnp.float32),
            pltpu.VMEM((bw // 2, _TW), jnp.float32),
            pltpu.VMEM((t1_len,), jnp.float32),
            pltpu.VMEM((t1_len,), jnp.float32),
            pltpu.SemaphoreType.DMA,
            pltpu.SemaphoreType.DMA,
            pltpu.SemaphoreType.DMA,
            pltpu.SemaphoreType.DMA,
            pltpu.SemaphoreType.DMA,
        ],
        compiler_params=pltpu.CompilerParams(
            use_tc_tiling_on_sc=False, needs_layout_passes=False
        ),
    )
    def gather_kernel(
        table_hbm, idx_hbm, out_hbm,
        idx_v, r0, r1, t0, t1,
        g0, g1, isem, s0, s1,
    ):
        wid = lax.axis_index("s") * _NUM_CORES + lax.axis_index("c")
        col0 = wid * bw
        rows = (r0, r1)
        gsem = (g0, g1)
        tiles = (t0, t1)
        ssem = (s0, s1)

        for h in range(hist):
            pltpu.async_copy(
                idx_hbm.at[pl.ds(h * batch + col0, bw)], idx_v.at[h], isem
            )
        for h in range(hist):
            pltpu.make_async_copy(
                idx_hbm.at[pl.ds(0, bw)], idx_v.at[0], isem
            ).wait()

        lane = lax.iota(jnp.int32, _LANES)
        cvec0 = lane * 128
        cvec1 = cvec0 + 2048
        hw = bw // 2

        def fire_gather(h, s, rb):
            pltpu.async_copy(
                table_hbm.at[idx_v.at[h].at[pl.ds(s * hw, hw)]], rows[rb], gsem[rb]
            )

        def wait_gather(rb):
            pltpu.make_async_copy(
                table_hbm.at[idx_v.at[0].at[pl.ds(0, hw)]], rows[rb], gsem[rb]
            ).wait()

        def transpose(rb, tb, s):
            src = rows[rb]
            dst = tiles[tb]

            def tr_body(i, carry):
                for j in range(8):
                    b = i * 8 + j
                    bg = s * hw + b
                    boff = (bg >> 7) * (ncg * 8 * 128) + (bg & 127)
                    v0 = src[b, pl.ds(0, _LANES)]
                    v1 = src[b, pl.ds(_LANES, _LANES)]
                    plsc.store_scatter(dst, [cvec0 + boff], v0)
                    plsc.store_scatter(dst, [cvec1 + boff], v1)
                return carry

            lax.fori_loop(0, hw // 8, tr_body, 0)

        def fire_stores(h, tb):
            for bt in range(nbt):
                for cg in range(ncg):
                    pltpu.async_copy(
                        tiles[tb].at[pl.ds((bt * ncg + cg) * 1024, 1024)],
                        out_hbm.at[
                            pl.ds(
                                h * slab + cg * (batch * 8)
                                + (wid * nbt + bt) * 1024,
                                1024,
                            )
                        ],
                        ssem[tb],
                    )

        def drain_stores(tb):
            for _ in range(nbt * ncg):
                pltpu.make_async_copy(
                    tiles[tb].at[pl.ds(0, 1024)],
                    out_hbm.at[pl.ds(0, 1024)],
                    ssem[tb],
                ).wait()

        for s in range(2):
            fire_gather(0, s, s)

        def step(h, s, tb, drain, fire):
            wait_gather(s)
            if s == 0 and drain:
                drain_stores(tb)
            transpose(s, tb, s)
            if s == 1:
                fire_stores(h, tb)
            if fire:
                fire_gather(h + 1, s, s)

        def pair(h0, drain0, drain1, fire0, fire1):
            for s in range(2):
                step(h0, s, 0, drain0, fire0)
            for s in range(2):
                step(h0 + 1, s, 1, drain1, fire1)

        pair(0, False, False, True, True)

        def body(i2, carry):
            pair(2 + i2 * 2, True, True, True, True)
            return carry

        n_main = (hist - 4) // 2
        lax.fori_loop(0, n_main, body, 0)

        pair(hist - 2, True, True, True, False)

        for tb in range(2):
            drain_stores(tb)

    return gather_kernel


def kernel(x, table):
    batch, hist = x.shape
    idx = x.T.reshape(batch * hist).astype(jnp.int32)
    tp = jnp.pad(table, ((0, 0), (0, _TW - _D)))
    flat = _make_gather(batch, hist)(tp, idx)
    a = flat.reshape(hist, _D // 8, batch // _TILE_B, 8, _TILE_B)
    return a.transpose(2, 4, 0, 1, 3).reshape(batch, hist, _D)
